# baseline (speedup 1.0000x reference)
import functools

import jax
import jax.numpy as jnp
import numpy as np
from jax import lax
from jax.experimental import pallas as pl
from jax.experimental.pallas import tpu as pltpu
from jax.experimental.pallas import tpu_sc as plsc

_NUM_SCALES = 16
_MAX_PARAMS = 2 ** 19
_FEATS = 2
_P1 = np.uint32(2654435761)
_P2 = np.uint32(805459861)

_NC, _NS = 2, 16
_NW = _NC * _NS
_C = 128
_GROUPS = _C // 16
_ROWS = _NUM_SCALES * 8 * _C


def _levels():
    b = np.exp((np.log(2048.0) - np.log(16.0)) / (_NUM_SCALES - 1))
    out = []
    for l in range(_NUM_SCALES):
        res = int(np.floor(16.0 * b ** l))
        dense = (res + 1) ** 3 <= _MAX_PARAMS
        out.append((res, dense, res + 1, (res + 1) ** 2))
    return out


_LEVELS = _levels()


def _corner_indices(l, res, dense, s1, s2, px, py, pz):
    base_off = l * _MAX_PARAMS
    idxs = []
    if dense:
        base = px + py * s1 + pz * s2 + base_off
        for c in range(8):
            ox, oy, oz = c & 1, (c >> 1) & 1, (c >> 2) & 1
            k = ox + oy * s1 + oz * s2
            idxs.append(base + k if k else base)
    else:
        hx0 = px.astype(jnp.uint32)
        hy0 = py.astype(jnp.uint32) * _P1
        hz0 = pz.astype(jnp.uint32) * _P2
        hx = (hx0, hx0 + jnp.uint32(1))
        hy = (hy0, hy0 + _P1)
        hz = (hz0, hz0 + _P2)
        lvl = jnp.uint32(base_off)
        msk = jnp.uint32(_MAX_PARAMS - 1)
        for c in range(8):
            ox, oy, oz = c & 1, (c >> 1) & 1, (c >> 2) & 1
            h = hx[ox] ^ hy[oy] ^ hz[oz]
            idxs.append(((h & msk) | lvl).astype(jnp.int32))
    return idxs


def _split_feats_tc(tab2d):
    total = tab2d.shape[0]
    bn = 32768
    grid = total // bn

    def body(i_ref, o0_ref, o1_ref):
        v = i_ref[...]
        o0_ref[...] = v[:, 0]
        o1_ref[...] = v[:, 1]

    return pl.pallas_call(
        body,
        grid=(grid,),
        in_specs=[pl.BlockSpec((bn, _FEATS), lambda i: (i, 0))],
        out_specs=[pl.BlockSpec((bn,), lambda i: (i,)),
                   pl.BlockSpec((bn,), lambda i: (i,))],
        out_shape=[jax.ShapeDtypeStruct((total,), jnp.float32),
                   jax.ShapeDtypeStruct((total,), jnp.float32)],
    )(tab2d)


def _transpose_tc(a, n):
    bn = 2048
    grid = (a.shape[1] + bn - 1) // bn

    def body(i_ref, o_ref):
        o_ref[...] = i_ref[...].T

    return pl.pallas_call(
        body,
        grid=(grid,),
        in_specs=[pl.BlockSpec((2 * _NUM_SCALES, bn), lambda i: (0, i))],
        out_specs=pl.BlockSpec((bn, 2 * _NUM_SCALES), lambda i: (i, 0)),
        out_shape=jax.ShapeDtypeStruct((n, 2 * _NUM_SCALES), jnp.float32),
    )(a)


def _make_kernel(n_pad):
    q_per_w = n_pad // _NW
    chunks = q_per_w // _C
    mesh = plsc.VectorSubcoreMesh(
        core_axis_name="c", subcore_axis_name="s",
        num_cores=_NC, num_subcores=_NS)

    @functools.partial(
        pl.kernel,
        out_type=jax.ShapeDtypeStruct((2 * _NUM_SCALES, n_pad), jnp.float32),
        mesh=mesh,
        scratch_types=[
            pltpu.VMEM((_C,), jnp.float32),
            pltpu.VMEM((_C,), jnp.float32),
            pltpu.VMEM((_C,), jnp.float32),
            pltpu.VMEM((_ROWS,), jnp.int32),
            pltpu.VMEM((_ROWS,), jnp.float32),
            pltpu.VMEM((_ROWS,), jnp.float32),
            pltpu.VMEM((2 * _NUM_SCALES, _C), jnp.float32),
            pltpu.SemaphoreType.DMA,
        ],
    )
    def kern(xx, yy, zz, tab0, tab1, out,
             xv, yv, zv, idxb, rows0, rows1, accb, sem):
        wid = lax.axis_index("s") * _NC + lax.axis_index("c")

        def chunk_body(i, carry):
            base = wid * q_per_w + i * _C
            pltpu.sync_copy(xx.at[pl.ds(base, _C)], xv)
            pltpu.sync_copy(yy.at[pl.ds(base, _C)], yv)
            pltpu.sync_copy(zz.at[pl.ds(base, _C)], zv)

            def phase1(g, carry1):
                o = g * 16
                xc = xv[pl.ds(o, 16)]
                yc = yv[pl.ds(o, 16)]
                zc = zv[pl.ds(o, 16)]
                for l, (res, dense, s1, s2) in enumerate(_LEVELS):
                    rf = jnp.float32(res)
                    px = (xc * rf).astype(jnp.int32)
                    py = (yc * rf).astype(jnp.int32)
                    pz = (zc * rf).astype(jnp.int32)
                    idxs = _corner_indices(l, res, dense, s1, s2, px, py, pz)
                    for c in range(8):
                        idxb[pl.ds(o + (l * 8 + c) * _C, 16)] = idxs[c]
                return carry1

            lax.fori_loop(0, _GROUPS, phase1, 0)

            cp0 = pltpu.async_copy(tab0.at[idxb], rows0, sem)
            cp1 = pltpu.async_copy(tab1.at[idxb], rows1, sem)
            cp0.wait()
            cp1.wait()

            def phase2(g, carry2):
                o = g * 16
                xc = xv[pl.ds(o, 16)]
                yc = yv[pl.ds(o, 16)]
                zc = zv[pl.ds(o, 16)]
                for l, (res, dense, s1, s2) in enumerate(_LEVELS):
                    rf = jnp.float32(res)
                    sx, sy, sz = xc * rf, yc * rf, zc * rf
                    px = sx.astype(jnp.int32)
                    py = sy.astype(jnp.int32)
                    pz = sz.astype(jnp.int32)
                    fx = sx - px.astype(jnp.float32)
                    fy = sy - py.astype(jnp.float32)
                    fz = sz - pz.astype(jnp.float32)
                    wx = (1.0 - fx, fx)
                    wy = (1.0 - fy, fy)
                    wz = (1.0 - fz, fz)
                    wxy = (wx[0] * wy[0], wx[1] * wy[0],
                           wx[0] * wy[1], wx[1] * wy[1])
                    acc0 = jnp.zeros((16,), jnp.float32)
                    acc1 = jnp.zeros((16,), jnp.float32)
                    for c in range(8):
                        ox, oy, oz = c & 1, (c >> 1) & 1, (c >> 2) & 1
                        offs = o + (l * 8 + c) * _C
                        g0 = rows0[pl.ds(offs, 16)]
                        g1 = rows1[pl.ds(offs, 16)]
                        w = wxy[oy * 2 + ox] * wz[oz]
                        acc0 = acc0 + w * g0
                        acc1 = acc1 + w * g1
                    accb[2 * l, pl.ds(o, 16)] = acc0
                    accb[2 * l + 1, pl.ds(o, 16)] = acc1
                return carry2

            lax.fori_loop(0, _GROUPS, phase2, 0)

            pltpu.sync_copy(accb, out.at[:, pl.ds(base, _C)])
            return carry

        lax.fori_loop(0, chunks, chunk_body, 0)

    return kern


def kernel(x, hash_table):
    n = x.shape[0]
    n_pad = ((n + _NW * _C - 1) // (_NW * _C)) * (_NW * _C)
    xp = jnp.pad(x, ((0, n_pad - n), (0, 0)))
    xx, yy, zz = xp[:, 0], xp[:, 1], xp[:, 2]
    tab2d = hash_table.reshape(_NUM_SCALES * _MAX_PARAMS, _FEATS)
    tab0, tab1 = _split_feats_tc(tab2d)
    out = _make_kernel(n_pad)(xx, yy, zz, tab0, tab1)
    return _transpose_tc(out, n)


# speedup vs baseline: 6.5787x; 6.5787x over previous
import functools

import jax
import jax.numpy as jnp
import numpy as np
from jax import lax
from jax.experimental import pallas as pl
from jax.experimental.pallas import tpu as pltpu
from jax.experimental.pallas import tpu_sc as plsc

_NUM_SCALES = 16
_MAX_PARAMS = 2 ** 19
_FEATS = 2
_P1 = np.uint32(2654435761)
_P2 = np.uint32(805459861)

_NC, _NS = 2, 16
_NW = _NC * _NS
_C = 64
_GROUPS = _C // 16
_ROWS = _NUM_SCALES * 8 * _C


def _levels():
    b = np.exp((np.log(2048.0) - np.log(16.0)) / (_NUM_SCALES - 1))
    out = []
    for l in range(_NUM_SCALES):
        res = int(np.floor(16.0 * b ** l))
        dense = (res + 1) ** 3 <= _MAX_PARAMS
        out.append((res, dense, res + 1, (res + 1) ** 2))
    return out


_LEVELS = _levels()


def _corner_indices(l, res, dense, s1, s2, px, py, pz):
    base_off = l * _MAX_PARAMS
    idxs = []
    if dense:
        base = px + py * s1 + pz * s2 + base_off
        for c in range(8):
            ox, oy, oz = c & 1, (c >> 1) & 1, (c >> 2) & 1
            k = ox + oy * s1 + oz * s2
            idxs.append(base + k if k else base)
    else:
        hx0 = px.astype(jnp.uint32)
        hy0 = py.astype(jnp.uint32) * _P1
        hz0 = pz.astype(jnp.uint32) * _P2
        hx = (hx0, hx0 + jnp.uint32(1))
        hy = (hy0, hy0 + _P1)
        hz = (hz0, hz0 + _P2)
        lvl = jnp.uint32(base_off)
        msk = jnp.uint32(_MAX_PARAMS - 1)
        for c in range(8):
            ox, oy, oz = c & 1, (c >> 1) & 1, (c >> 2) & 1
            h = hx[ox] ^ hy[oy] ^ hz[oz]
            idxs.append(((h & msk) | lvl).astype(jnp.int32))
    return idxs


def _split_feats_tc(tab2d):
    total = tab2d.shape[0]
    bn = 32768
    grid = total // bn

    def body(i_ref, o0_ref, o1_ref):
        v = i_ref[...]
        o0_ref[...] = v[:, 0]
        o1_ref[...] = v[:, 1]

    return pl.pallas_call(
        body,
        grid=(grid,),
        in_specs=[pl.BlockSpec((bn, _FEATS), lambda i: (i, 0))],
        out_specs=[pl.BlockSpec((bn,), lambda i: (i,)),
                   pl.BlockSpec((bn,), lambda i: (i,))],
        out_shape=[jax.ShapeDtypeStruct((total,), jnp.float32),
                   jax.ShapeDtypeStruct((total,), jnp.float32)],
    )(tab2d)


def _transpose_tc(a, n):
    bn = 2048
    grid = (a.shape[1] + bn - 1) // bn

    def body(i_ref, o_ref):
        o_ref[...] = i_ref[...].T

    return pl.pallas_call(
        body,
        grid=(grid,),
        in_specs=[pl.BlockSpec((2 * _NUM_SCALES, bn), lambda i: (0, i))],
        out_specs=pl.BlockSpec((bn, 2 * _NUM_SCALES), lambda i: (i, 0)),
        out_shape=jax.ShapeDtypeStruct((n, 2 * _NUM_SCALES), jnp.float32),
    )(a)


def _make_kernel(n_pad):
    q_per_w = n_pad // _NW
    chunks = q_per_w // _C
    assert chunks % 2 == 0
    half = chunks // 2
    mesh = plsc.VectorSubcoreMesh(
        core_axis_name="c", subcore_axis_name="s",
        num_cores=_NC, num_subcores=_NS)

    def _buf():
        return [
            pltpu.VMEM((_C,), jnp.float32),
            pltpu.VMEM((_C,), jnp.float32),
            pltpu.VMEM((_C,), jnp.float32),
            pltpu.VMEM((_ROWS,), jnp.int32),
            pltpu.VMEM((_ROWS,), jnp.float32),
            pltpu.VMEM((_ROWS,), jnp.float32),
            pltpu.SemaphoreType.DMA,
        ]

    @functools.partial(
        pl.kernel,
        out_type=jax.ShapeDtypeStruct((2 * _NUM_SCALES, n_pad), jnp.float32),
        mesh=mesh,
        scratch_types=_buf() + _buf()
        + [pltpu.VMEM((2 * _NUM_SCALES, 2 * _C), jnp.float32)],
    )
    def kern(xx, yy, zz, tab0, tab1, out,
             xva, yva, zva, idxa, r0a, r1a, sema,
             xvb, yvb, zvb, idxb, r0b, r1b, semb, accp):
        wid = lax.axis_index("s") * _NC + lax.axis_index("c")
        bufs = ((xva, yva, zva, idxa, r0a, r1a, sema),
                (xvb, yvb, zvb, idxb, r0b, r1b, semb))

        def launch(i, b):
            xv, yv, zv, idx, r0, r1, sem = b
            base = wid * q_per_w + i * _C
            pltpu.sync_copy(xx.at[pl.ds(base, _C)], xv)
            pltpu.sync_copy(yy.at[pl.ds(base, _C)], yv)
            pltpu.sync_copy(zz.at[pl.ds(base, _C)], zv)

            def phase1(g, carry1):
                o = g * 16
                xc = xv[pl.ds(o, 16)]
                yc = yv[pl.ds(o, 16)]
                zc = zv[pl.ds(o, 16)]
                for l, (res, dense, s1, s2) in enumerate(_LEVELS):
                    rf = jnp.float32(res)
                    px = (xc * rf).astype(jnp.int32)
                    py = (yc * rf).astype(jnp.int32)
                    pz = (zc * rf).astype(jnp.int32)
                    idxs = _corner_indices(l, res, dense, s1, s2, px, py, pz)
                    for c in range(8):
                        idx[pl.ds(o + (l * 8 + c) * _C, 16)] = idxs[c]
                return carry1

            lax.fori_loop(0, _GROUPS, phase1, 0)
            cp0 = pltpu.async_copy(tab0.at[idx], r0, sem)
            cp1 = pltpu.async_copy(tab1.at[idx], r1, sem)
            return cp0, cp1

        def finish(i, b, handles, half_off):
            xv, yv, zv, idx, r0, r1, sem = b
            handles[0].wait()
            handles[1].wait()

            def phase2(g, carry2):
                o = g * 16
                xc = xv[pl.ds(o, 16)]
                yc = yv[pl.ds(o, 16)]
                zc = zv[pl.ds(o, 16)]
                for l, (res, dense, s1, s2) in enumerate(_LEVELS):
                    rf = jnp.float32(res)
                    sx, sy, sz = xc * rf, yc * rf, zc * rf
                    px = sx.astype(jnp.int32)
                    py = sy.astype(jnp.int32)
                    pz = sz.astype(jnp.int32)
                    fx = sx - px.astype(jnp.float32)
                    fy = sy - py.astype(jnp.float32)
                    fz = sz - pz.astype(jnp.float32)
                    wx = (1.0 - fx, fx)
                    wy = (1.0 - fy, fy)
                    wz = (1.0 - fz, fz)
                    wxy = (wx[0] * wy[0], wx[1] * wy[0],
                           wx[0] * wy[1], wx[1] * wy[1])
                    acc0 = jnp.zeros((16,), jnp.float32)
                    acc1 = jnp.zeros((16,), jnp.float32)
                    for c in range(8):
                        ox, oy, oz = c & 1, (c >> 1) & 1, (c >> 2) & 1
                        offs = o + (l * 8 + c) * _C
                        g0 = r0[pl.ds(offs, 16)]
                        g1 = r1[pl.ds(offs, 16)]
                        w = wxy[oy * 2 + ox] * wz[oz]
                        acc0 = acc0 + w * g0
                        acc1 = acc1 + w * g1
                    accp[2 * l, pl.ds(half_off + o, 16)] = acc0
                    accp[2 * l + 1, pl.ds(half_off + o, 16)] = acc1
                return carry2

            lax.fori_loop(0, _GROUPS, phase2, 0)

        def pair_body(j, carry):
            ha = launch(2 * j, bufs[0])
            hb = launch(2 * j + 1, bufs[1])
            finish(2 * j, bufs[0], ha, 0)
            finish(2 * j + 1, bufs[1], hb, _C)
            base = wid * q_per_w + 2 * j * _C
            pltpu.sync_copy(accp, out.at[:, pl.ds(base, 2 * _C)])
            return carry

        lax.fori_loop(0, half, pair_body, 0)

    return kern


def kernel(x, hash_table):
    n = x.shape[0]
    cpw = (n + _NW * _C - 1) // (_NW * _C)
    cpw += cpw % 2
    n_pad = cpw * _NW * _C
    xp = jnp.pad(x, ((0, n_pad - n), (0, 0)))
    xx, yy, zz = xp[:, 0], xp[:, 1], xp[:, 2]
    one = 1.0 + 0.0 * x[0, 0]
    tab0 = (hash_table[:, :, 0] * one).reshape(_NUM_SCALES * _MAX_PARAMS)
    tab1 = (hash_table[:, :, 1] * one).reshape(_NUM_SCALES * _MAX_PARAMS)
    out = _make_kernel(n_pad)(xx, yy, zz, tab0, tab1)
    return _transpose_tc(out, n)
